# baseline (speedup 1.0000x reference)
import functools

import jax
import jax.numpy as jnp
from jax.experimental import pallas as pl
from jax.experimental.pallas import tpu as pltpu

B, S, D = 1, 2048, 768
H, DH = 12, 64
CBS = 32
SBS = 32
NSEL = 4
SW = 128
MLP = 3072
NB = S // CBS
SCALE = DH ** -0.5
NEG = -1e30

TQ = 512
NQT = S // TQ
ATQ = 512
HG = 6
HP = H // HG
WS = ATQ + 256

F32 = jnp.float32
BF16 = jnp.bfloat16


def _ln_body(xt, g, b):
    mu = jnp.mean(xt, axis=-1, keepdims=True)
    xc = xt - mu
    var = jnp.mean(xc * xc, axis=-1, keepdims=True)
    return xc * jax.lax.rsqrt(var + 1e-5) * g + b


def _dot(a, b):
    return jnp.dot(a.astype(BF16), b.astype(BF16), preferred_element_type=F32)


def _dot_tlhs(a, b, prefer=F32):
    return jax.lax.dot_general(a.astype(BF16), b.astype(BF16),
                               (((0,), (0,)), ((), ())),
                               preferred_element_type=prefer)


def _dot_trhs(a, b):
    return jax.lax.dot_general(a.astype(BF16), b.astype(BF16),
                               (((1,), (1,)), ((), ())),
                               preferred_element_type=F32)


def _k1(x_ref, g_ref, b_ref, wq_ref, wk_ref, wv_ref, wg_ref,
        q_ref, k_ref, v_ref, gates_ref, wqb_ref, wkb_ref, wvb_ref):
    @pl.when(pl.program_id(0) == 0)
    def _():
        wqb_ref[...] = wq_ref[...].astype(BF16)
        wkb_ref[...] = wk_ref[...].astype(BF16)
        wvb_ref[...] = wv_ref[...].astype(BF16)
    xn = _ln_body(x_ref[...], g_ref[...], b_ref[...]).astype(BF16)
    q_ref[...] = jnp.dot(xn, wqb_ref[...],
                         preferred_element_type=F32).astype(BF16)
    k_ref[...] = jnp.dot(xn, wkb_ref[...],
                         preferred_element_type=F32).astype(BF16)
    v_ref[...] = jnp.dot(xn, wvb_ref[...],
                         preferred_element_type=F32).astype(BF16)
    gates_ref[...] = _dot(xn, wg_ref[...])


def _proj(x, ln_g, ln_b, Wq, Wk, Wv, Wg):
    return pl.pallas_call(
        _k1,
        grid=(NQT,),
        in_specs=[
            pl.BlockSpec((TQ, D), lambda i: (i, 0)),
            pl.BlockSpec((1, D), lambda i: (0, 0)),
            pl.BlockSpec((1, D), lambda i: (0, 0)),
            pl.BlockSpec((D, D), lambda i: (0, 0)),
            pl.BlockSpec((D, D), lambda i: (0, 0)),
            pl.BlockSpec((D, D), lambda i: (0, 0)),
            pl.BlockSpec((D, 3 * H), lambda i: (0, 0)),
        ],
        out_specs=[
            pl.BlockSpec((TQ, D), lambda i: (i, 0)),
            pl.BlockSpec((TQ, D), lambda i: (i, 0)),
            pl.BlockSpec((TQ, D), lambda i: (i, 0)),
            pl.BlockSpec((TQ, 3 * H), lambda i: (i, 0)),
        ],
        out_shape=[
            jax.ShapeDtypeStruct((S, D), BF16),
            jax.ShapeDtypeStruct((S, D), BF16),
            jax.ShapeDtypeStruct((S, D), BF16),
            jax.ShapeDtypeStruct((S, 3 * H), F32),
        ],
        scratch_shapes=[
            pltpu.VMEM((D, D), BF16),
            pltpu.VMEM((D, D), BF16),
            pltpu.VMEM((D, D), BF16),
        ],
    )(x, ln_g, ln_b, Wq, Wk, Wv, Wg)


def _k3(q_ref, k_ref, v_ref, g_ref, wck_ref, wcv_ref, o_ref, *, ti, kw, nbk):
    row0 = ti * ATQ
    q2 = q_ref[...]
    k2 = k_ref[...]
    v2 = v_ref[...]
    n_i = jax.lax.broadcasted_iota(jnp.int32, (nbk, kw), 0)
    s_i = jax.lax.broadcasted_iota(jnp.int32, (nbk, kw), 1)
    Ex = jnp.where(s_i // CBS == n_i, 1.0, 0.0).astype(BF16)
    posT = jax.lax.broadcasted_iota(jnp.int32, (nbk, ATQ), 1) + row0
    blkT = jax.lax.broadcasted_iota(jnp.int32, (nbk, ATQ), 0)
    cmaskT = (blkT + 1) * CBS - 1 <= posT
    if ti == 0:
        row = jax.lax.broadcasted_iota(jnp.int32, (ATQ, kw), 0)
        col = jax.lax.broadcasted_iota(jnp.int32, (ATQ, kw), 1)
        causal = col <= row
        wmask = causal & (col > row - SW)
    else:
        colw = jax.lax.broadcasted_iota(jnp.int32, (ATQ, WS), 1) + row0 - 256
        roww = jax.lax.broadcasted_iota(jnp.int32, (ATQ, WS), 0) + row0
        wmask = (colw <= roww) & (colw > roww - SW)
    onescol = (jax.lax.broadcasted_iota(jnp.int32, (kw, DH), 1) == 0
               ).astype(BF16)
    gsig = jax.nn.sigmoid(g_ref[...])
    outs = []
    for hh in range(HG):
        lo, hi = hh * DH, (hh + 1) * DH
        q = q2[:, lo:hi] * jnp.asarray(SCALE, BF16)
        k = k2[:, lo:hi]
        v = v2[:, lo:hi]
        s = _dot_trhs(q, k)
        c = jnp.max(s, axis=-1, keepdims=True)
        e = jnp.exp(s - c)
        kc = _dot(_dot(Ex, k) * (1.0 / CBS), wck_ref[...])
        vc = _dot(_dot(Ex, v) * (1.0 / CBS), wcv_ref[...])
        scT = _dot_trhs(kc, q)
        scmT = jnp.where(cmaskT, scT, NEG)
        mT = jnp.max(scmT, axis=0, keepdims=True)
        eT = jnp.exp(scmT - mT)
        pcT = eT / jnp.sum(eT, axis=0, keepdims=True)
        pcT = jnp.where(posT[:1] >= CBS - 1, pcT, 0.0)
        o_cmp = _dot_tlhs(pcT, vc)
        impT = jnp.where(cmaskT, pcT, -1.0)
        selT = jnp.zeros((nbk, ATQ), jnp.bool_)
        for _ in range(NSEL):
            mx = jnp.max(impT, axis=0, keepdims=True)
            ismax = impT == mx
            first = jnp.min(jnp.where(ismax, blkT, nbk), axis=0, keepdims=True)
            onehot = blkT == first
            selT = selT | onehot
            impT = jnp.where(onehot, -2.0, impT)
        msel = _dot_tlhs(selT.astype(BF16), Ex)
        es = e * msel
        if ti == 0:
            es = jnp.where(causal, es, 0.0)
        es = es.astype(BF16)
        vv = jnp.concatenate([v, onescol], axis=1)
        oz = jnp.dot(es, vv, preferred_element_type=F32)
        o_sel = oz[:, :DH] / oz[:, DH:DH + 1]
        if ti == 0:
            ew = jnp.where(wmask, e, 0.0).astype(BF16)
            wz = jnp.dot(ew, vv, preferred_element_type=F32)
        else:
            kslab = k[row0 - 256:row0 + ATQ]
            vslab = v[row0 - 256:row0 + ATQ]
            sslab = _dot_trhs(q, kslab)
            ew = jnp.where(wmask, jnp.exp(sslab - c), 0.0).astype(BF16)
            vvs = jnp.concatenate([vslab, onescol[:WS]], axis=1)
            wz = jnp.dot(ew, vvs, preferred_element_type=F32)
        o_win = wz[:, :DH] / wz[:, DH:DH + 1]
        g = gsig[hh]
        outs.append((g[:, 0:1] * o_cmp + g[:, 1:2] * o_sel
                     + g[:, 2:3] * o_win).astype(BF16))
    o_ref[...] = jnp.concatenate(outs, axis=1)


def _attention_tile(q, k, v, gates_h, Wck, Wcv, ti):
    kw = (ti + 1) * ATQ
    nbk = kw // CBS
    body = functools.partial(_k3, ti=ti, kw=kw, nbk=nbk)
    return pl.pallas_call(
        body,
        grid=(HP,),
        in_specs=[
            pl.BlockSpec((ATQ, HG * DH), lambda h: (ti, h)),
            pl.BlockSpec((kw, HG * DH), lambda h: (0, h)),
            pl.BlockSpec((kw, HG * DH), lambda h: (0, h)),
            pl.BlockSpec((HG, ATQ, 3), lambda h: (h, ti, 0)),
            pl.BlockSpec((DH, DH), lambda h: (0, 0)),
            pl.BlockSpec((DH, DH), lambda h: (0, 0)),
        ],
        out_specs=pl.BlockSpec((ATQ, HG * DH), lambda h: (0, h)),
        out_shape=jax.ShapeDtypeStruct((ATQ, D), BF16),
    )(q, k, v, gates_h, Wck, Wcv)


def _k45(o_ref, x_ref, wo_ref, g_ref, b_ref, w1_ref, b1_ref, w2_ref, b2_ref,
         y_ref, wob_ref, w1b_ref, w2b_ref):
    @pl.when(pl.program_id(0) == 0)
    def _():
        wob_ref[...] = wo_ref[...].astype(BF16)
        w1b_ref[...] = w1_ref[...].astype(BF16)
        w2b_ref[...] = w2_ref[...].astype(BF16)
    x1 = x_ref[...] + jnp.dot(o_ref[...], wob_ref[...],
                              preferred_element_type=F32)
    xn = _ln_body(x1, g_ref[...], b_ref[...]).astype(BF16)
    hgelu = jax.nn.gelu(jnp.dot(xn, w1b_ref[...], preferred_element_type=F32)
                        + b1_ref[...])
    y_ref[...] = x1 + jnp.dot(hgelu.astype(BF16), w2b_ref[...],
                              preferred_element_type=F32) + b2_ref[...]


def _tail(o, x, Wo, ln_g, ln_b, W1, b1, W2, b2):
    return pl.pallas_call(
        _k45,
        grid=(NQT,),
        in_specs=[
            pl.BlockSpec((TQ, D), lambda i: (i, 0)),
            pl.BlockSpec((TQ, D), lambda i: (i, 0)),
            pl.BlockSpec((D, D), lambda i: (0, 0)),
            pl.BlockSpec((1, D), lambda i: (0, 0)),
            pl.BlockSpec((1, D), lambda i: (0, 0)),
            pl.BlockSpec((D, MLP), lambda i: (0, 0)),
            pl.BlockSpec((1, MLP), lambda i: (0, 0)),
            pl.BlockSpec((MLP, D), lambda i: (0, 0)),
            pl.BlockSpec((1, D), lambda i: (0, 0)),
        ],
        out_specs=pl.BlockSpec((TQ, D), lambda i: (i, 0)),
        out_shape=jax.ShapeDtypeStruct((S, D), F32),
        scratch_shapes=[
            pltpu.VMEM((D, D), BF16),
            pltpu.VMEM((D, MLP), BF16),
            pltpu.VMEM((MLP, D), BF16),
        ],
    )(o, x, Wo, ln_g, ln_b, W1, b1, W2, b2)


@jax.jit
def _run(x, ln1_g, ln1_b, Wq, Wk, Wv, Wck, Wcv, Wg, Wo, ln2_g, ln2_b, W1, b1, W2, b2):
    x2 = x[0]
    q, k, v, gates = _proj(x2, ln1_g[None], ln1_b[None], Wq, Wk, Wv,
                           Wg.astype(BF16))
    gates_h = gates.reshape(S, H, 3).transpose(1, 0, 2)
    Wckb = Wck.astype(BF16)
    Wcvb = Wcv.astype(BF16)
    o = jnp.concatenate(
        [_attention_tile(q, k, v, gates_h, Wckb, Wcvb, ti) for ti in range(4)],
        axis=0)
    y = _tail(o, x2, Wo, ln2_g[None], ln2_b[None], W1, b1[None], W2, b2[None])
    return y[None]


def kernel(x, ln1_g, ln1_b, Wq, Wk, Wv, Wck, Wcv, Wg, Wo, ln2_g, ln2_b, W1, b1, W2, b2):
    return _run(x, ln1_g, ln1_b, Wq, Wk, Wv, Wck, Wcv, Wg, Wo,
                ln2_g, ln2_b, W1, b1, W2, b2)


# speedup vs baseline: 1.0033x; 1.0033x over previous
import functools

import jax
import jax.numpy as jnp
from jax.experimental import pallas as pl
from jax.experimental.pallas import tpu as pltpu

B, S, D = 1, 2048, 768
H, DH = 12, 64
CBS = 32
SBS = 32
NSEL = 4
SW = 128
MLP = 3072
NB = S // CBS
SCALE = DH ** -0.5
NEG = -1e30

TQ = 512
NQT = S // TQ
ATQ = 512
HG = 4
HP = H // HG
WS = ATQ + 256

F32 = jnp.float32
BF16 = jnp.bfloat16


def _ln_body(xt, g, b):
    mu = jnp.mean(xt, axis=-1, keepdims=True)
    xc = xt - mu
    var = jnp.mean(xc * xc, axis=-1, keepdims=True)
    return xc * jax.lax.rsqrt(var + 1e-5) * g + b


def _dot(a, b):
    return jnp.dot(a.astype(BF16), b.astype(BF16), preferred_element_type=F32)


def _dot_tlhs(a, b, prefer=F32):
    return jax.lax.dot_general(a.astype(BF16), b.astype(BF16),
                               (((0,), (0,)), ((), ())),
                               preferred_element_type=prefer)


def _dot_trhs(a, b):
    return jax.lax.dot_general(a.astype(BF16), b.astype(BF16),
                               (((1,), (1,)), ((), ())),
                               preferred_element_type=F32)


def _k1(x_ref, g_ref, b_ref, wq_ref, wk_ref, wv_ref, wg_ref,
        q_ref, k_ref, v_ref, gates_ref, wqb_ref, wkb_ref, wvb_ref):
    @pl.when(pl.program_id(0) == 0)
    def _():
        wqb_ref[...] = wq_ref[...].astype(BF16)
        wkb_ref[...] = wk_ref[...].astype(BF16)
        wvb_ref[...] = wv_ref[...].astype(BF16)
    xn = _ln_body(x_ref[...], g_ref[...], b_ref[...]).astype(BF16)
    q_ref[...] = jnp.dot(xn, wqb_ref[...],
                         preferred_element_type=F32).astype(BF16)
    k_ref[...] = jnp.dot(xn, wkb_ref[...],
                         preferred_element_type=F32).astype(BF16)
    v_ref[...] = jnp.dot(xn, wvb_ref[...],
                         preferred_element_type=F32).astype(BF16)
    gates_ref[...] = _dot(xn, wg_ref[...])


def _proj(x, ln_g, ln_b, Wq, Wk, Wv, Wg):
    return pl.pallas_call(
        _k1,
        grid=(NQT,),
        in_specs=[
            pl.BlockSpec((TQ, D), lambda i: (i, 0)),
            pl.BlockSpec((1, D), lambda i: (0, 0)),
            pl.BlockSpec((1, D), lambda i: (0, 0)),
            pl.BlockSpec((D, D), lambda i: (0, 0)),
            pl.BlockSpec((D, D), lambda i: (0, 0)),
            pl.BlockSpec((D, D), lambda i: (0, 0)),
            pl.BlockSpec((D, 3 * H), lambda i: (0, 0)),
        ],
        out_specs=[
            pl.BlockSpec((TQ, D), lambda i: (i, 0)),
            pl.BlockSpec((TQ, D), lambda i: (i, 0)),
            pl.BlockSpec((TQ, D), lambda i: (i, 0)),
            pl.BlockSpec((TQ, 3 * H), lambda i: (i, 0)),
        ],
        out_shape=[
            jax.ShapeDtypeStruct((S, D), BF16),
            jax.ShapeDtypeStruct((S, D), BF16),
            jax.ShapeDtypeStruct((S, D), BF16),
            jax.ShapeDtypeStruct((S, 3 * H), F32),
        ],
        scratch_shapes=[
            pltpu.VMEM((D, D), BF16),
            pltpu.VMEM((D, D), BF16),
            pltpu.VMEM((D, D), BF16),
        ],
    )(x, ln_g, ln_b, Wq, Wk, Wv, Wg)


def _k3(q_ref, k_ref, v_ref, g_ref, wck_ref, wcv_ref, o_ref, *, ti, kw, nbk):
    row0 = ti * ATQ
    q2 = q_ref[...]
    k2 = k_ref[...]
    v2 = v_ref[...]
    n_i = jax.lax.broadcasted_iota(jnp.int32, (nbk, kw), 0)
    s_i = jax.lax.broadcasted_iota(jnp.int32, (nbk, kw), 1)
    Ex = jnp.where(s_i // CBS == n_i, 1.0, 0.0).astype(BF16)
    posT = jax.lax.broadcasted_iota(jnp.int32, (nbk, ATQ), 1) + row0
    blkT = jax.lax.broadcasted_iota(jnp.int32, (nbk, ATQ), 0)
    cmaskT = (blkT + 1) * CBS - 1 <= posT
    if ti == 0:
        row = jax.lax.broadcasted_iota(jnp.int32, (ATQ, kw), 0)
        col = jax.lax.broadcasted_iota(jnp.int32, (ATQ, kw), 1)
        causal = col <= row
        wmask = causal & (col > row - SW)
    else:
        colw = jax.lax.broadcasted_iota(jnp.int32, (ATQ, WS), 1) + row0 - 256
        roww = jax.lax.broadcasted_iota(jnp.int32, (ATQ, WS), 0) + row0
        wmask = (colw <= roww) & (colw > roww - SW)
    onescol = (jax.lax.broadcasted_iota(jnp.int32, (kw, DH), 1) == 0
               ).astype(BF16)
    gsig = jax.nn.sigmoid(g_ref[...])
    outs = []
    for hh in range(HG):
        lo, hi = hh * DH, (hh + 1) * DH
        q = q2[:, lo:hi] * jnp.asarray(SCALE, BF16)
        k = k2[:, lo:hi]
        v = v2[:, lo:hi]
        s = _dot_trhs(q, k)
        c = jnp.max(s, axis=-1, keepdims=True)
        e = jnp.exp(s - c)
        kc = _dot(_dot(Ex, k) * (1.0 / CBS), wck_ref[...])
        vc = _dot(_dot(Ex, v) * (1.0 / CBS), wcv_ref[...])
        scT = _dot_trhs(kc, q)
        scmT = jnp.where(cmaskT, scT, NEG)
        mT = jnp.max(scmT, axis=0, keepdims=True)
        eT = jnp.exp(scmT - mT)
        pcT = eT / jnp.sum(eT, axis=0, keepdims=True)
        pcT = jnp.where(posT[:1] >= CBS - 1, pcT, 0.0)
        o_cmp = _dot_tlhs(pcT, vc)
        impT = jnp.where(cmaskT, pcT, -1.0)
        selT = jnp.zeros((nbk, ATQ), jnp.bool_)
        for _ in range(NSEL):
            mx = jnp.max(impT, axis=0, keepdims=True)
            ismax = impT == mx
            first = jnp.min(jnp.where(ismax, blkT, nbk), axis=0, keepdims=True)
            onehot = blkT == first
            selT = selT | onehot
            impT = jnp.where(onehot, -2.0, impT)
        msel = _dot_tlhs(selT.astype(BF16), Ex)
        es = e * msel
        if ti == 0:
            es = jnp.where(causal, es, 0.0)
        es = es.astype(BF16)
        vv = jnp.concatenate([v, onescol], axis=1)
        oz = jnp.dot(es, vv, preferred_element_type=F32)
        o_sel = oz[:, :DH] / oz[:, DH:DH + 1]
        if ti == 0:
            ew = jnp.where(wmask, e, 0.0).astype(BF16)
            wz = jnp.dot(ew, vv, preferred_element_type=F32)
        else:
            kslab = k[row0 - 256:row0 + ATQ]
            vslab = v[row0 - 256:row0 + ATQ]
            sslab = _dot_trhs(q, kslab)
            ew = jnp.where(wmask, jnp.exp(sslab - c), 0.0).astype(BF16)
            vvs = jnp.concatenate([vslab, onescol[:WS]], axis=1)
            wz = jnp.dot(ew, vvs, preferred_element_type=F32)
        o_win = wz[:, :DH] / wz[:, DH:DH + 1]
        g = gsig[hh]
        outs.append((g[:, 0:1] * o_cmp + g[:, 1:2] * o_sel
                     + g[:, 2:3] * o_win).astype(BF16))
    o_ref[...] = jnp.concatenate(outs, axis=1)


def _attention_tile(q, k, v, gates_h, Wck, Wcv, ti):
    kw = (ti + 1) * ATQ
    nbk = kw // CBS
    body = functools.partial(_k3, ti=ti, kw=kw, nbk=nbk)
    return pl.pallas_call(
        body,
        grid=(HP,),
        in_specs=[
            pl.BlockSpec((ATQ, HG * DH), lambda h: (ti, h)),
            pl.BlockSpec((kw, HG * DH), lambda h: (0, h)),
            pl.BlockSpec((kw, HG * DH), lambda h: (0, h)),
            pl.BlockSpec((HG, ATQ, 3), lambda h: (h, ti, 0)),
            pl.BlockSpec((DH, DH), lambda h: (0, 0)),
            pl.BlockSpec((DH, DH), lambda h: (0, 0)),
        ],
        out_specs=pl.BlockSpec((ATQ, HG * DH), lambda h: (0, h)),
        out_shape=jax.ShapeDtypeStruct((ATQ, D), BF16),
    )(q, k, v, gates_h, Wck, Wcv)


def _k45(o_ref, x_ref, wo_ref, g_ref, b_ref, w1_ref, b1_ref, w2_ref, b2_ref,
         y_ref, wob_ref, w1b_ref, w2b_ref):
    @pl.when(pl.program_id(0) == 0)
    def _():
        wob_ref[...] = wo_ref[...].astype(BF16)
        w1b_ref[...] = w1_ref[...].astype(BF16)
        w2b_ref[...] = w2_ref[...].astype(BF16)
    x1 = x_ref[...] + jnp.dot(o_ref[...], wob_ref[...],
                              preferred_element_type=F32)
    xn = _ln_body(x1, g_ref[...], b_ref[...]).astype(BF16)
    hgelu = jax.nn.gelu(jnp.dot(xn, w1b_ref[...], preferred_element_type=F32)
                        + b1_ref[...])
    y_ref[...] = x1 + jnp.dot(hgelu.astype(BF16), w2b_ref[...],
                              preferred_element_type=F32) + b2_ref[...]


def _tail(o, x, Wo, ln_g, ln_b, W1, b1, W2, b2):
    return pl.pallas_call(
        _k45,
        grid=(NQT,),
        in_specs=[
            pl.BlockSpec((TQ, D), lambda i: (i, 0)),
            pl.BlockSpec((TQ, D), lambda i: (i, 0)),
            pl.BlockSpec((D, D), lambda i: (0, 0)),
            pl.BlockSpec((1, D), lambda i: (0, 0)),
            pl.BlockSpec((1, D), lambda i: (0, 0)),
            pl.BlockSpec((D, MLP), lambda i: (0, 0)),
            pl.BlockSpec((1, MLP), lambda i: (0, 0)),
            pl.BlockSpec((MLP, D), lambda i: (0, 0)),
            pl.BlockSpec((1, D), lambda i: (0, 0)),
        ],
        out_specs=pl.BlockSpec((TQ, D), lambda i: (i, 0)),
        out_shape=jax.ShapeDtypeStruct((S, D), F32),
        scratch_shapes=[
            pltpu.VMEM((D, D), BF16),
            pltpu.VMEM((D, MLP), BF16),
            pltpu.VMEM((MLP, D), BF16),
        ],
    )(o, x, Wo, ln_g, ln_b, W1, b1, W2, b2)


@jax.jit
def _run(x, ln1_g, ln1_b, Wq, Wk, Wv, Wck, Wcv, Wg, Wo, ln2_g, ln2_b, W1, b1, W2, b2):
    x2 = x[0]
    q, k, v, gates = _proj(x2, ln1_g[None], ln1_b[None], Wq, Wk, Wv,
                           Wg.astype(BF16))
    gates_h = gates.reshape(S, H, 3).transpose(1, 0, 2)
    Wckb = Wck.astype(BF16)
    Wcvb = Wcv.astype(BF16)
    o = jnp.concatenate(
        [_attention_tile(q, k, v, gates_h, Wckb, Wcvb, ti) for ti in range(4)],
        axis=0)
    y = _tail(o, x2, Wo, ln2_g[None], ln2_b[None], W1, b1[None], W2, b2[None])
    return y[None]


def kernel(x, ln1_g, ln1_b, Wq, Wk, Wv, Wck, Wcv, Wg, Wo, ln2_g, ln2_b, W1, b1, W2, b2):
    return _run(x, ln1_g, ln1_b, Wq, Wk, Wv, Wck, Wcv, Wg, Wo,
                ln2_g, ln2_b, W1, b1, W2, b2)
